# Initial kernel scaffold; baseline (speedup 1.0000x reference)
import jax
import jax.numpy as jnp
from jax.experimental import pallas as pl


def kernel(x, W, ema_cluster_size, ema_w):
    raise NotImplementedError("write your pallas kernel here")


# speedup vs baseline: 6.1390x; 6.1390x over previous
import functools

import jax
import jax.numpy as jnp
from jax import lax
from jax.experimental import pallas as pl
from jax.experimental.pallas import tpu as pltpu
from jax.experimental.pallas import tpu_sc as plsc

K_EMB = 8192
D_EMB = 256
N_TOK = 8192
DECAY = 0.99
EPSILON = 1e-05
COMMITMENT_COST = 0.25


TM = 512
TK = 1024
NT = N_TOK // TM
NK = K_EMB // TK


def _argmin_body(fsq_ref, wsq_ref, flat_ref, w_ref, idx_out, bestv, besti):
    kblk = pl.program_id(1)

    @pl.when(kblk == 0)
    def _():
        bestv[...] = jnp.full((TM, 1), jnp.inf, jnp.float32)
        besti[...] = jnp.zeros((TM, 1), jnp.int32)

    m = lax.dot_general(flat_ref[...], w_ref[...], (((1,), (1,)), ((), ())),
                        preferred_element_type=jnp.float32)
    d = (fsq_ref[...] + wsq_ref[...]) - 2.0 * m
    rowmin = jnp.min(d, axis=1, keepdims=True)
    cols = lax.broadcasted_iota(jnp.int32, (TM, TK), 1)
    rowarg = jnp.min(jnp.where(d == rowmin, cols, TK), axis=1, keepdims=True)
    rowarg = rowarg + kblk * TK
    better = rowmin < bestv[...]
    besti[...] = jnp.where(better, rowarg, besti[...])
    bestv[...] = jnp.where(better, rowmin, bestv[...])

    @pl.when(kblk == NK - 1)
    def _():
        idx_out[...] = besti[...]


def _stage_a(fsq, wsq2d, flat, W):
    return pl.pallas_call(
        _argmin_body,
        grid=(NT, NK),
        in_specs=[
            pl.BlockSpec((TM, 1), lambda i, k: (i, 0)),
            pl.BlockSpec((1, TK), lambda i, k: (0, k)),
            pl.BlockSpec((TM, D_EMB), lambda i, k: (i, 0)),
            pl.BlockSpec((TK, D_EMB), lambda i, k: (k, 0)),
        ],
        out_specs=pl.BlockSpec((TM, 1), lambda i, k: (i, 0)),
        out_shape=jax.ShapeDtypeStruct((N_TOK, 1), jnp.int32),
        scratch_shapes=[pltpu.VMEM((TM, 1), jnp.float32),
                        pltpu.VMEM((TM, 1), jnp.int32)],
        compiler_params=pltpu.CompilerParams(
            dimension_semantics=("arbitrary", "arbitrary")),
    )(fsq, wsq2d, flat, W)


NC = 2
NS = 16
TPW = N_TOK // NS
CHUNK = 128
NCHUNK = TPW // CHUNK
CNT_R = K_EMB // 128


def _scatter_body(flat2_hbm, idxr_hbm, z128_hbm, iota_hbm,
                  dw_hbm, cnt_hbm, idx_v, rows_c, cnt_local, iota_v,
                  dw_sh, cnt_sh, _sem):
    c = lax.axis_index("c")
    s = lax.axis_index("s")
    base = s * TPW
    pltpu.sync_copy(z128_hbm.at[pl.ds(base, TPW)], dw_sh.at[pl.ds(base, TPW)])
    pltpu.sync_copy(z128_hbm.at[pl.ds(0, CNT_R)], cnt_local)
    pltpu.sync_copy(idxr_hbm.at[pl.ds(s * NCHUNK, NCHUNK)], idx_v)
    pltpu.sync_copy(iota_hbm, iota_v)

    @pl.when(s == 0)
    def _():
        pltpu.sync_copy(z128_hbm.at[pl.ds(0, CNT_R)], cnt_sh)

    ones16 = jnp.ones((16,), jnp.float32)
    for j in range(NCHUNK):
        for i in range(CHUNK // 16):
            v = idx_v[j, pl.ds(i * 16, 16)]
            hi = lax.shift_right_logical(v, 7)
            lo = lax.bitwise_and(v, 127)
            plsc.addupdate_scatter(cnt_local, [hi, lo], ones16)
    plsc.subcore_barrier()
    for j in range(NCHUNK):
        pltpu.sync_copy(
            flat2_hbm.at[pl.ds(c * N_TOK + base + j * CHUNK, CHUNK)], rows_c)
        pltpu.sync_copy(rows_c, dw_sh.at[idx_v.at[j]], add=True)
    pltpu.sync_copy(cnt_local, cnt_sh.at[iota_v], add=True)
    plsc.subcore_barrier()
    pltpu.sync_copy(dw_sh.at[pl.ds(base, TPW)],
                    dw_hbm.at[pl.ds(c * K_EMB + base, TPW)])
    rpw = CNT_R // (NC * NS)
    pltpu.sync_copy(cnt_sh.at[pl.ds(c * (CNT_R // NC) + s * rpw, rpw)],
                    cnt_hbm.at[pl.ds(c * (CNT_R // NC) + s * rpw, rpw)])


def _stage_b(flat2, idx_rows, z128, iota):
    mesh = plsc.VectorSubcoreMesh(core_axis_name="c", subcore_axis_name="s")
    f = functools.partial(
        pl.kernel,
        out_type=(jax.ShapeDtypeStruct((NC * K_EMB, 128), jnp.float32),
                  jax.ShapeDtypeStruct((CNT_R, 128), jnp.float32)),
        mesh=mesh,
        scratch_types=[
            pltpu.VMEM((NCHUNK, CHUNK), jnp.int32),
            pltpu.VMEM((CHUNK, 128), jnp.float32),
            pltpu.VMEM((CNT_R, 128), jnp.float32),
            pltpu.VMEM((CNT_R,), jnp.int32),
            pltpu.VMEM_SHARED((K_EMB, 128), jnp.float32),
            pltpu.VMEM_SHARED((CNT_R, 128), jnp.float32),
            pltpu.SemaphoreType.DMA,
        ],
        compiler_params=pltpu.CompilerParams(needs_layout_passes=False),
    )(_scatter_body)
    return f(flat2, idx_rows, z128, iota)


RB = 512
NRB = K_EMB // RB


def _wnew_body(cnt_ref, ecs_ref, dw0_ref, dw1_ref, emaw_ref, wnew_ref, csn_ref):
    i = pl.program_id(0)

    @pl.when(i == 0)
    def _():
        cs = ecs_ref[...] * DECAY + cnt_ref[...] * (1.0 - DECAY)
        n = jnp.sum(cs)
        csn_ref[...] = (cs + EPSILON) / (n + K_EMB * EPSILON) * n

    dw = jnp.concatenate([dw0_ref[...], dw1_ref[...]], axis=1)
    ema_w_new = emaw_ref[...] * DECAY + dw * (1.0 - DECAY)
    wnew_ref[...] = ema_w_new / csn_ref[pl.ds(i * RB, RB), :]


def _stage_c(cnt2d, ecs2d, dw_halves, ema_w):
    return pl.pallas_call(
        _wnew_body,
        grid=(NRB,),
        in_specs=[
            pl.BlockSpec((K_EMB, 1), lambda i: (0, 0)),
            pl.BlockSpec((K_EMB, 1), lambda i: (0, 0)),
            pl.BlockSpec((RB, 128), lambda i: (i, 0)),
            pl.BlockSpec((RB, 128), lambda i: (NRB + i, 0)),
            pl.BlockSpec((RB, D_EMB), lambda i: (i, 0)),
        ],
        out_specs=pl.BlockSpec((RB, D_EMB), lambda i: (i, 0)),
        out_shape=jax.ShapeDtypeStruct((K_EMB, D_EMB), jnp.float32),
        scratch_shapes=[pltpu.VMEM((K_EMB, 1), jnp.float32)],
        compiler_params=pltpu.CompilerParams(
            dimension_semantics=("arbitrary",)),
    )(cnt2d, ecs2d, dw_halves, dw_halves, ema_w)


GPW = N_TOK // (NC * NS)
GCHUNK = GPW // CHUNK


def _gather_body(wnew_hbm, idxr_hbm, q_hbm, idx_v, rows_v, sem):
    c = lax.axis_index("c")
    s = lax.axis_index("s")
    wid = s * NC + c
    pltpu.sync_copy(idxr_hbm.at[pl.ds(wid * GCHUNK, GCHUNK)], idx_v)
    for j in range(GCHUNK):
        pltpu.async_copy(wnew_hbm.at[idx_v.at[j]],
                         rows_v.at[pl.ds(j * CHUNK, CHUNK)], sem).wait()
    pltpu.sync_copy(rows_v, q_hbm.at[pl.ds(wid * GPW, GPW)])


def _stage_d(wnew, idx_rows):
    mesh = plsc.VectorSubcoreMesh(core_axis_name="c", subcore_axis_name="s")
    f = functools.partial(
        pl.kernel,
        out_type=jax.ShapeDtypeStruct((N_TOK, D_EMB), jnp.float32),
        mesh=mesh,
        scratch_types=[
            pltpu.VMEM((GCHUNK, CHUNK), jnp.int32),
            pltpu.VMEM((GPW, D_EMB), jnp.float32),
            pltpu.SemaphoreType.DMA,
        ],
    )(_gather_body)
    return f(wnew, idx_rows)


def _loss_body(q_ref, x_ref, qst_ref, loss_ref):
    q = q_ref[...]
    xx = x_ref[...]
    d = q - xx
    qst_ref[...] = xx + d
    loss_ref[0, 0] = jnp.sum(d * d) * (COMMITMENT_COST / (N_TOK * D_EMB))


def _stage_e(qflat, flat):
    return pl.pallas_call(
        _loss_body,
        out_specs=[
            pl.BlockSpec(memory_space=pltpu.VMEM),
            pl.BlockSpec(memory_space=pltpu.SMEM),
        ],
        out_shape=[
            jax.ShapeDtypeStruct((N_TOK, D_EMB), jnp.float32),
            jax.ShapeDtypeStruct((1, 1), jnp.float32),
        ],
        compiler_params=pltpu.CompilerParams(
            vmem_limit_bytes=100 * 1024 * 1024),
    )(qflat, flat)


def kernel(x, W, ema_cluster_size, ema_w):
    inputs = jnp.transpose(x, (0, 2, 3, 1))
    input_shape = inputs.shape
    flat = inputs.reshape(N_TOK, D_EMB)
    fsq = jnp.sum(flat ** 2, axis=1, keepdims=True)
    wsq2d = jnp.sum(W ** 2, axis=1)[None, :]

    idx2d = _stage_a(fsq, wsq2d, flat, W)
    idx_rows = idx2d.reshape(N_TOK // CHUNK, CHUNK)

    flat2 = jnp.concatenate([flat[:, :128], flat[:, 128:]], axis=0)
    z128 = jnp.zeros((K_EMB, 128), jnp.float32)
    iota = jnp.arange(CNT_R, dtype=jnp.int32)
    dw_halves, cnt = _stage_b(flat2, idx_rows, z128, iota)

    wnew = _stage_c(cnt.reshape(K_EMB, 1), ema_cluster_size.reshape(K_EMB, 1),
                    dw_halves, ema_w)
    qflat = _stage_d(wnew, idx_rows)
    qst, loss11 = _stage_e(qflat, flat)

    quantized = jnp.transpose(qst.reshape(input_shape), (0, 3, 1, 2))
    return (loss11[0, 0], quantized, idx2d)


# speedup vs baseline: 7.6513x; 1.2463x over previous
import functools

import jax
import jax.numpy as jnp
from jax import lax
from jax.experimental import pallas as pl
from jax.experimental.pallas import tpu as pltpu
from jax.experimental.pallas import tpu_sc as plsc

K_EMB = 8192
D_EMB = 256
N_TOK = 8192
DECAY = 0.99
EPSILON = 1e-05
COMMITMENT_COST = 0.25


TM = 1024
TK = 2048
NT = N_TOK // TM
NK = K_EMB // TK


def _argmin_body(fsq_ref, wsq_ref, flat_ref, w_ref, idx_out, bestv, besti):
    kblk = pl.program_id(1)

    @pl.when(kblk == 0)
    def _():
        bestv[...] = jnp.full((TM, 1), jnp.inf, jnp.float32)
        besti[...] = jnp.zeros((TM, 1), jnp.int32)

    m = lax.dot_general(flat_ref[...], w_ref[...], (((1,), (1,)), ((), ())),
                        preferred_element_type=jnp.float32)
    d = (fsq_ref[...] + wsq_ref[...]) - 2.0 * m
    rowmin = jnp.min(d, axis=1, keepdims=True)
    cols = lax.broadcasted_iota(jnp.int32, (TM, TK), 1)
    rowarg = jnp.min(jnp.where(d == rowmin, cols, TK), axis=1, keepdims=True)
    rowarg = rowarg + kblk * TK
    better = rowmin < bestv[...]
    besti[...] = jnp.where(better, rowarg, besti[...])
    bestv[...] = jnp.where(better, rowmin, bestv[...])

    @pl.when(kblk == NK - 1)
    def _():
        idx_out[...] = besti[...]


def _stage_a(fsq, wsq2d, flat, W):
    return pl.pallas_call(
        _argmin_body,
        grid=(NT, NK),
        in_specs=[
            pl.BlockSpec((TM, 1), lambda i, k: (i, 0)),
            pl.BlockSpec((1, TK), lambda i, k: (0, k)),
            pl.BlockSpec((TM, D_EMB), lambda i, k: (i, 0)),
            pl.BlockSpec((TK, D_EMB), lambda i, k: (k, 0)),
        ],
        out_specs=pl.BlockSpec((TM, 1), lambda i, k: (i, 0)),
        out_shape=jax.ShapeDtypeStruct((N_TOK, 1), jnp.int32),
        scratch_shapes=[pltpu.VMEM((TM, 1), jnp.float32),
                        pltpu.VMEM((TM, 1), jnp.int32)],
        compiler_params=pltpu.CompilerParams(
            dimension_semantics=("arbitrary", "arbitrary")),
    )(fsq, wsq2d, flat, W)


NC = 2
NS = 16
TPW = N_TOK // NS
CHUNK = 128
NCHUNK = TPW // CHUNK
CNT_R = K_EMB // 128


def _scatter_body(flat2_hbm, idxr_hbm, z128_hbm, iota_hbm,
                  dw_hbm, cnt_hbm, idx_v, rows_c, cnt_local, iota_v,
                  dw_sh, cnt_sh, _sem):
    c = lax.axis_index("c")
    s = lax.axis_index("s")
    base = s * TPW
    pltpu.sync_copy(z128_hbm.at[pl.ds(base, TPW)], dw_sh.at[pl.ds(base, TPW)])
    pltpu.sync_copy(z128_hbm.at[pl.ds(0, CNT_R)], cnt_local)
    pltpu.sync_copy(idxr_hbm.at[pl.ds(s * NCHUNK, NCHUNK)], idx_v)
    pltpu.sync_copy(iota_hbm, iota_v)

    @pl.when(s == 0)
    def _():
        pltpu.sync_copy(z128_hbm.at[pl.ds(0, CNT_R)], cnt_sh)

    ones16 = jnp.ones((16,), jnp.float32)
    for j in range(NCHUNK):
        for i in range(CHUNK // 16):
            v = idx_v[j, pl.ds(i * 16, 16)]
            hi = lax.shift_right_logical(v, 7)
            lo = lax.bitwise_and(v, 127)
            plsc.addupdate_scatter(cnt_local, [hi, lo], ones16)
    plsc.subcore_barrier()
    for j in range(NCHUNK):
        pltpu.sync_copy(
            flat2_hbm.at[pl.ds(c * N_TOK + base + j * CHUNK, CHUNK)], rows_c)
        pltpu.sync_copy(rows_c, dw_sh.at[idx_v.at[j]], add=True)
    pltpu.sync_copy(cnt_local, cnt_sh.at[iota_v], add=True)
    plsc.subcore_barrier()
    pltpu.sync_copy(dw_sh.at[pl.ds(base, TPW)],
                    dw_hbm.at[pl.ds(c * K_EMB + base, TPW)])
    rpw = CNT_R // (NC * NS)
    pltpu.sync_copy(cnt_sh.at[pl.ds(c * (CNT_R // NC) + s * rpw, rpw)],
                    cnt_hbm.at[pl.ds(c * (CNT_R // NC) + s * rpw, rpw)])


def _stage_b(flat2, idx_rows, z128, iota):
    mesh = plsc.VectorSubcoreMesh(core_axis_name="c", subcore_axis_name="s")
    f = functools.partial(
        pl.kernel,
        out_type=(jax.ShapeDtypeStruct((NC * K_EMB, 128), jnp.float32),
                  jax.ShapeDtypeStruct((CNT_R, 128), jnp.float32)),
        mesh=mesh,
        scratch_types=[
            pltpu.VMEM((NCHUNK, CHUNK), jnp.int32),
            pltpu.VMEM((CHUNK, 128), jnp.float32),
            pltpu.VMEM((CNT_R, 128), jnp.float32),
            pltpu.VMEM((CNT_R,), jnp.int32),
            pltpu.VMEM_SHARED((K_EMB, 128), jnp.float32),
            pltpu.VMEM_SHARED((CNT_R, 128), jnp.float32),
            pltpu.SemaphoreType.DMA,
        ],
        compiler_params=pltpu.CompilerParams(needs_layout_passes=False),
    )(_scatter_body)
    return f(flat2, idx_rows, z128, iota)


RB = 512
NRB = K_EMB // RB


def _wnew_body(cnt_ref, ecs_ref, dw0_ref, dw1_ref, emaw_ref, wnew_ref, csn_ref):
    i = pl.program_id(0)

    @pl.when(i == 0)
    def _():
        cs = ecs_ref[...] * DECAY + cnt_ref[...] * (1.0 - DECAY)
        n = jnp.sum(cs)
        csn_ref[...] = (cs + EPSILON) / (n + K_EMB * EPSILON) * n

    dw = jnp.concatenate([dw0_ref[...], dw1_ref[...]], axis=1)
    ema_w_new = emaw_ref[...] * DECAY + dw * (1.0 - DECAY)
    wnew_ref[...] = ema_w_new / csn_ref[pl.ds(i * RB, RB), :]


def _stage_c(cnt2d, ecs2d, dw_halves, ema_w):
    return pl.pallas_call(
        _wnew_body,
        grid=(NRB,),
        in_specs=[
            pl.BlockSpec((K_EMB, 1), lambda i: (0, 0)),
            pl.BlockSpec((K_EMB, 1), lambda i: (0, 0)),
            pl.BlockSpec((RB, 128), lambda i: (i, 0)),
            pl.BlockSpec((RB, 128), lambda i: (NRB + i, 0)),
            pl.BlockSpec((RB, D_EMB), lambda i: (i, 0)),
        ],
        out_specs=pl.BlockSpec((RB, D_EMB), lambda i: (i, 0)),
        out_shape=jax.ShapeDtypeStruct((K_EMB, D_EMB), jnp.float32),
        scratch_shapes=[pltpu.VMEM((K_EMB, 1), jnp.float32)],
        compiler_params=pltpu.CompilerParams(
            dimension_semantics=("arbitrary",)),
    )(cnt2d, ecs2d, dw_halves, dw_halves, ema_w)


GPW = N_TOK // (NC * NS)
GCHUNK = GPW // CHUNK


def _gather_body(wnew_hbm, idxr_hbm, q_hbm, idx_v, rows_v, sem):
    c = lax.axis_index("c")
    s = lax.axis_index("s")
    wid = s * NC + c
    pltpu.sync_copy(idxr_hbm.at[pl.ds(wid * GCHUNK, GCHUNK)], idx_v)
    for j in range(GCHUNK):
        pltpu.async_copy(wnew_hbm.at[idx_v.at[j]],
                         rows_v.at[pl.ds(j * CHUNK, CHUNK)], sem).wait()
    pltpu.sync_copy(rows_v, q_hbm.at[pl.ds(wid * GPW, GPW)])


def _stage_d(wnew, idx_rows):
    mesh = plsc.VectorSubcoreMesh(core_axis_name="c", subcore_axis_name="s")
    f = functools.partial(
        pl.kernel,
        out_type=jax.ShapeDtypeStruct((N_TOK, D_EMB), jnp.float32),
        mesh=mesh,
        scratch_types=[
            pltpu.VMEM((GCHUNK, CHUNK), jnp.int32),
            pltpu.VMEM((GPW, D_EMB), jnp.float32),
            pltpu.SemaphoreType.DMA,
        ],
    )(_gather_body)
    return f(wnew, idx_rows)


def _loss_body(q_ref, x_ref, qst_ref, loss_ref):
    q = q_ref[...]
    xx = x_ref[...]
    d = q - xx
    qst_ref[...] = xx + d
    loss_ref[0, 0] = jnp.sum(d * d) * (COMMITMENT_COST / (N_TOK * D_EMB))


def _stage_e(qflat, flat):
    return pl.pallas_call(
        _loss_body,
        out_specs=[
            pl.BlockSpec(memory_space=pltpu.VMEM),
            pl.BlockSpec(memory_space=pltpu.SMEM),
        ],
        out_shape=[
            jax.ShapeDtypeStruct((N_TOK, D_EMB), jnp.float32),
            jax.ShapeDtypeStruct((1, 1), jnp.float32),
        ],
        compiler_params=pltpu.CompilerParams(
            vmem_limit_bytes=100 * 1024 * 1024),
    )(qflat, flat)


def kernel(x, W, ema_cluster_size, ema_w):
    inputs = jnp.transpose(x, (0, 2, 3, 1))
    input_shape = inputs.shape
    flat = inputs.reshape(N_TOK, D_EMB)
    fsq = jnp.sum(flat ** 2, axis=1, keepdims=True)
    wsq2d = jnp.sum(W ** 2, axis=1)[None, :]

    idx2d = _stage_a(fsq, wsq2d, flat, W)
    idx_rows = idx2d.reshape(N_TOK // CHUNK, CHUNK)

    flat2 = jnp.concatenate([flat[:, :128], flat[:, 128:]], axis=0)
    z128 = jnp.zeros((K_EMB, 128), jnp.float32)
    iota = jnp.arange(CNT_R, dtype=jnp.int32)
    dw_halves, cnt = _stage_b(flat2, idx_rows, z128, iota)

    wnew = _stage_c(cnt.reshape(K_EMB, 1), ema_cluster_size.reshape(K_EMB, 1),
                    dw_halves, ema_w)
    qflat = _stage_d(wnew, idx_rows)
    qst, loss11 = _stage_e(qflat, flat)

    quantized = jnp.transpose(qst.reshape(input_shape), (0, 3, 1, 2))
    return (loss11[0, 0], quantized, idx2d)


# speedup vs baseline: 7.8304x; 1.0234x over previous
import functools

import jax
import jax.numpy as jnp
from jax import lax
from jax.experimental import pallas as pl
from jax.experimental.pallas import tpu as pltpu
from jax.experimental.pallas import tpu_sc as plsc

K_EMB = 8192
D_EMB = 256
N_TOK = 8192
DECAY = 0.99
EPSILON = 1e-05
COMMITMENT_COST = 0.25


TM = 2048
TK = 2048
NT = N_TOK // TM
NK = K_EMB // TK


def _argmin_body(fsq_ref, wsq_ref, flat_ref, w_ref, idx_out, bestv, besti):
    kblk = pl.program_id(1)

    @pl.when(kblk == 0)
    def _():
        bestv[...] = jnp.full((TM, 1), jnp.inf, jnp.float32)
        besti[...] = jnp.zeros((TM, 1), jnp.int32)

    m = lax.dot_general(flat_ref[...], w_ref[...], (((1,), (1,)), ((), ())),
                        preferred_element_type=jnp.float32)
    d = (fsq_ref[...] + wsq_ref[...]) - 2.0 * m
    rowmin = jnp.min(d, axis=1, keepdims=True)
    cols = lax.broadcasted_iota(jnp.int32, (TM, TK), 1)
    rowarg = jnp.min(jnp.where(d == rowmin, cols, TK), axis=1, keepdims=True)
    rowarg = rowarg + kblk * TK
    better = rowmin < bestv[...]
    besti[...] = jnp.where(better, rowarg, besti[...])
    bestv[...] = jnp.where(better, rowmin, bestv[...])

    @pl.when(kblk == NK - 1)
    def _():
        idx_out[...] = besti[...]


def _stage_a(fsq, wsq2d, flat, W):
    return pl.pallas_call(
        _argmin_body,
        grid=(NT, NK),
        in_specs=[
            pl.BlockSpec((TM, 1), lambda i, k: (i, 0)),
            pl.BlockSpec((1, TK), lambda i, k: (0, k)),
            pl.BlockSpec((TM, D_EMB), lambda i, k: (i, 0)),
            pl.BlockSpec((TK, D_EMB), lambda i, k: (k, 0)),
        ],
        out_specs=pl.BlockSpec((TM, 1), lambda i, k: (i, 0)),
        out_shape=jax.ShapeDtypeStruct((N_TOK, 1), jnp.int32),
        scratch_shapes=[pltpu.VMEM((TM, 1), jnp.float32),
                        pltpu.VMEM((TM, 1), jnp.int32)],
        compiler_params=pltpu.CompilerParams(
            dimension_semantics=("arbitrary", "arbitrary")),
    )(fsq, wsq2d, flat, W)


NC = 2
NS = 16
TPW = N_TOK // NS
CHUNK = 128
NCHUNK = TPW // CHUNK
CNT_R = K_EMB // 128


def _scatter_body(flat2_hbm, idxr_hbm, z128_hbm, iota_hbm,
                  dw_hbm, cnt_hbm, idx_v, rows_c, cnt_local, iota_v,
                  dw_sh, cnt_sh, _sem):
    c = lax.axis_index("c")
    s = lax.axis_index("s")
    base = s * TPW
    pltpu.sync_copy(z128_hbm.at[pl.ds(base, TPW)], dw_sh.at[pl.ds(base, TPW)])
    pltpu.sync_copy(z128_hbm.at[pl.ds(0, CNT_R)], cnt_local)
    pltpu.sync_copy(idxr_hbm.at[pl.ds(s * NCHUNK, NCHUNK)], idx_v)
    pltpu.sync_copy(iota_hbm, iota_v)

    @pl.when(s == 0)
    def _():
        pltpu.sync_copy(z128_hbm.at[pl.ds(0, CNT_R)], cnt_sh)

    ones16 = jnp.ones((16,), jnp.float32)
    for j in range(NCHUNK):
        for i in range(CHUNK // 16):
            v = idx_v[j, pl.ds(i * 16, 16)]
            hi = lax.shift_right_logical(v, 7)
            lo = lax.bitwise_and(v, 127)
            plsc.addupdate_scatter(cnt_local, [hi, lo], ones16)
    plsc.subcore_barrier()
    for j in range(NCHUNK):
        pltpu.sync_copy(
            flat2_hbm.at[pl.ds(c * N_TOK + base + j * CHUNK, CHUNK)], rows_c)
        pltpu.sync_copy(rows_c, dw_sh.at[idx_v.at[j]], add=True)
    pltpu.sync_copy(cnt_local, cnt_sh.at[iota_v], add=True)
    plsc.subcore_barrier()
    pltpu.sync_copy(dw_sh.at[pl.ds(base, TPW)],
                    dw_hbm.at[pl.ds(c * K_EMB + base, TPW)])
    rpw = CNT_R // (NC * NS)
    pltpu.sync_copy(cnt_sh.at[pl.ds(c * (CNT_R // NC) + s * rpw, rpw)],
                    cnt_hbm.at[pl.ds(c * (CNT_R // NC) + s * rpw, rpw)])


def _stage_b(flat2, idx_rows, z128, iota):
    mesh = plsc.VectorSubcoreMesh(core_axis_name="c", subcore_axis_name="s")
    f = functools.partial(
        pl.kernel,
        out_type=(jax.ShapeDtypeStruct((NC * K_EMB, 128), jnp.float32),
                  jax.ShapeDtypeStruct((CNT_R, 128), jnp.float32)),
        mesh=mesh,
        scratch_types=[
            pltpu.VMEM((NCHUNK, CHUNK), jnp.int32),
            pltpu.VMEM((CHUNK, 128), jnp.float32),
            pltpu.VMEM((CNT_R, 128), jnp.float32),
            pltpu.VMEM((CNT_R,), jnp.int32),
            pltpu.VMEM_SHARED((K_EMB, 128), jnp.float32),
            pltpu.VMEM_SHARED((CNT_R, 128), jnp.float32),
            pltpu.SemaphoreType.DMA,
        ],
        compiler_params=pltpu.CompilerParams(needs_layout_passes=False),
    )(_scatter_body)
    return f(flat2, idx_rows, z128, iota)


RB = 512
NRB = K_EMB // RB


def _wnew_body(cnt_ref, ecs_ref, dw0_ref, dw1_ref, emaw_ref, wnew_ref, csn_ref):
    i = pl.program_id(0)

    @pl.when(i == 0)
    def _():
        cs = ecs_ref[...] * DECAY + cnt_ref[...] * (1.0 - DECAY)
        n = jnp.sum(cs)
        csn_ref[...] = (cs + EPSILON) / (n + K_EMB * EPSILON) * n

    dw = jnp.concatenate([dw0_ref[...], dw1_ref[...]], axis=1)
    ema_w_new = emaw_ref[...] * DECAY + dw * (1.0 - DECAY)
    wnew_ref[...] = ema_w_new / csn_ref[pl.ds(i * RB, RB), :]


def _stage_c(cnt2d, ecs2d, dw_halves, ema_w):
    return pl.pallas_call(
        _wnew_body,
        grid=(NRB,),
        in_specs=[
            pl.BlockSpec((K_EMB, 1), lambda i: (0, 0)),
            pl.BlockSpec((K_EMB, 1), lambda i: (0, 0)),
            pl.BlockSpec((RB, 128), lambda i: (i, 0)),
            pl.BlockSpec((RB, 128), lambda i: (NRB + i, 0)),
            pl.BlockSpec((RB, D_EMB), lambda i: (i, 0)),
        ],
        out_specs=pl.BlockSpec((RB, D_EMB), lambda i: (i, 0)),
        out_shape=jax.ShapeDtypeStruct((K_EMB, D_EMB), jnp.float32),
        scratch_shapes=[pltpu.VMEM((K_EMB, 1), jnp.float32)],
        compiler_params=pltpu.CompilerParams(
            dimension_semantics=("arbitrary",)),
    )(cnt2d, ecs2d, dw_halves, dw_halves, ema_w)


GPW = N_TOK // (NC * NS)
GCHUNK = GPW // CHUNK


def _gather_body(wnew_hbm, idxr_hbm, q_hbm, idx_v, rows_v, sem):
    c = lax.axis_index("c")
    s = lax.axis_index("s")
    wid = s * NC + c
    pltpu.sync_copy(idxr_hbm.at[pl.ds(wid * GCHUNK, GCHUNK)], idx_v)
    for j in range(GCHUNK):
        pltpu.async_copy(wnew_hbm.at[idx_v.at[j]],
                         rows_v.at[pl.ds(j * CHUNK, CHUNK)], sem).wait()
    pltpu.sync_copy(rows_v, q_hbm.at[pl.ds(wid * GPW, GPW)])


def _stage_d(wnew, idx_rows):
    mesh = plsc.VectorSubcoreMesh(core_axis_name="c", subcore_axis_name="s")
    f = functools.partial(
        pl.kernel,
        out_type=jax.ShapeDtypeStruct((N_TOK, D_EMB), jnp.float32),
        mesh=mesh,
        scratch_types=[
            pltpu.VMEM((GCHUNK, CHUNK), jnp.int32),
            pltpu.VMEM((GPW, D_EMB), jnp.float32),
            pltpu.SemaphoreType.DMA,
        ],
    )(_gather_body)
    return f(wnew, idx_rows)


def _loss_body(q_ref, x_ref, qst_ref, loss_ref):
    q = q_ref[...]
    xx = x_ref[...]
    d = q - xx
    qst_ref[...] = xx + d
    loss_ref[0, 0] = jnp.sum(d * d) * (COMMITMENT_COST / (N_TOK * D_EMB))


def _stage_e(qflat, flat):
    return pl.pallas_call(
        _loss_body,
        out_specs=[
            pl.BlockSpec(memory_space=pltpu.VMEM),
            pl.BlockSpec(memory_space=pltpu.SMEM),
        ],
        out_shape=[
            jax.ShapeDtypeStruct((N_TOK, D_EMB), jnp.float32),
            jax.ShapeDtypeStruct((1, 1), jnp.float32),
        ],
        compiler_params=pltpu.CompilerParams(
            vmem_limit_bytes=100 * 1024 * 1024),
    )(qflat, flat)


def kernel(x, W, ema_cluster_size, ema_w):
    inputs = jnp.transpose(x, (0, 2, 3, 1))
    input_shape = inputs.shape
    flat = inputs.reshape(N_TOK, D_EMB)
    fsq = jnp.sum(flat ** 2, axis=1, keepdims=True)
    wsq2d = jnp.sum(W ** 2, axis=1)[None, :]

    idx2d = _stage_a(fsq, wsq2d, flat, W)
    idx_rows = idx2d.reshape(N_TOK // CHUNK, CHUNK)

    flat2 = jnp.concatenate([flat[:, :128], flat[:, 128:]], axis=0)
    z128 = jnp.zeros((K_EMB, 128), jnp.float32)
    iota = jnp.arange(CNT_R, dtype=jnp.int32)
    dw_halves, cnt = _stage_b(flat2, idx_rows, z128, iota)

    wnew = _stage_c(cnt.reshape(K_EMB, 1), ema_cluster_size.reshape(K_EMB, 1),
                    dw_halves, ema_w)
    qflat = _stage_d(wnew, idx_rows)
    qst, loss11 = _stage_e(qflat, flat)

    quantized = jnp.transpose(qst.reshape(input_shape), (0, 3, 1, 2))
    return (loss11[0, 0], quantized, idx2d)
